# baseline (speedup 1.0000x reference)
import functools

import jax
import jax.numpy as jnp
from jax import lax
from jax.experimental import pallas as pl
from jax.experimental.pallas import tpu as pltpu
from jax.experimental.pallas import tpu_sc as plsc

N = 10000
NPAD = 10240
STRIPE = NPAD // 16
NW = 32
CH = 96
CPW = 108
EPW = CPW * CH
ETOT = NW * EPW
ETOT2 = ETOT + 2 * CH
HIGH = lax.Precision.HIGHEST


def _dense_front(xp, W, As16, Ad16):
    BLK = 512

    def body(x_ref, w_ref, as_ref, ad_ref, h_ref, ts_ref, td_ref):
        h = jnp.dot(x_ref[...], w_ref[...], precision=HIGH,
                    preferred_element_type=jnp.float32)
        h_ref[...] = h
        ts_ref[...] = jnp.dot(h, as_ref[...], precision=HIGH,
                              preferred_element_type=jnp.float32)
        td_ref[...] = jnp.dot(h, ad_ref[...], precision=HIGH,
                              preferred_element_type=jnp.float32)

    return pl.pallas_call(
        body,
        grid=(NPAD // BLK,),
        in_specs=[
            pl.BlockSpec((BLK, 128), lambda i: (i, 0)),
            pl.BlockSpec((128, 128), lambda i: (0, 0)),
            pl.BlockSpec((128, 16), lambda i: (0, 0)),
            pl.BlockSpec((128, 16), lambda i: (0, 0)),
        ],
        out_specs=[
            pl.BlockSpec((BLK, 128), lambda i: (i, 0)),
            pl.BlockSpec((BLK, 16), lambda i: (i, 0)),
            pl.BlockSpec((BLK, 16), lambda i: (i, 0)),
        ],
        out_shape=[
            jax.ShapeDtypeStruct((NPAD, 128), jnp.float32),
            jax.ShapeDtypeStruct((NPAD, 16), jnp.float32),
            jax.ShapeDtypeStruct((NPAD, 16), jnp.float32),
        ],
    )(xp, W, As16, Ad16)


def _dense_finish(onum, oden, S, b2d, do_elu):
    BLK = 512

    def body(n_ref, d_ref, s_ref, b_ref, o_ref):
        num = n_ref[0] + n_ref[1]
        den16 = d_ref[0] + d_ref[1]
        den = jnp.dot(den16, s_ref[...], precision=HIGH,
                      preferred_element_type=jnp.float32)
        o = num / (den + 1e-16) + b_ref[...]
        if do_elu:
            o = jnp.where(o > 0.0, o, jnp.exp(o) - 1.0)
        o_ref[...] = o

    return pl.pallas_call(
        body,
        grid=(NPAD // BLK,),
        in_specs=[
            pl.BlockSpec((2, BLK, 128), lambda i: (0, i, 0)),
            pl.BlockSpec((2, BLK, 16), lambda i: (0, i, 0)),
            pl.BlockSpec((16, 128), lambda i: (0, 0)),
            pl.BlockSpec((1, 128), lambda i: (0, 0)),
        ],
        out_specs=pl.BlockSpec((BLK, 128), lambda i: (i, 0)),
        out_shape=jax.ShapeDtypeStruct((NPAD, 128), jnp.float32),
    )(onum, oden, S, b2d)


def _sc_edge_pass(h, ts, td, srcp, dstp):
    mesh = plsc.VectorSubcoreMesh(core_axis_name="c", subcore_axis_name="s")

    @functools.partial(
        pl.kernel,
        compiler_params=pltpu.CompilerParams(
            needs_layout_passes=False, use_tc_tiling_on_sc=False),
        out_type=(
            jax.ShapeDtypeStruct((2, NPAD, 128), jnp.float32),
            jax.ShapeDtypeStruct((2, NPAD, 16), jnp.float32),
        ),
        mesh=mesh,
        scratch_types=[
            pltpu.VMEM((2, 2, CH), jnp.int32),
            pltpu.VMEM((2, CH, 128), jnp.float32),
            pltpu.VMEM((2, CH, 16), jnp.float32),
            pltpu.VMEM((2, CH, 16), jnp.float32),
            pltpu.VMEM((CH, 16), jnp.float32),
            pltpu.VMEM_SHARED((NPAD, 128), jnp.float32),
            pltpu.VMEM_SHARED((NPAD, 16), jnp.float32),
            pltpu.SemaphoreType.DMA,
            pltpu.SemaphoreType.DMA,
            pltpu.SemaphoreType.DMA,
            pltpu.SemaphoreType.DMA,
        ],
    )
    def k(h_hbm, ts_hbm, td_hbm, src_hbm, dst_hbm, on_hbm, od_hbm,
          idxb, hrows, tsr, tdr, wrows, accn, accd,
          isem0, isem1, gsem0, gsem1):
        cid = lax.axis_index("c")
        sid = lax.axis_index("s")
        wid = cid * 16 + sid
        base = wid * EPW
        zero16 = jnp.zeros((16,), jnp.float32)
        isem = (isem0, isem1)
        gsem = (gsem0, gsem1)

        def idx_copies(p, e0):
            return (pltpu.make_async_copy(
                        src_hbm.at[pl.ds(e0, CH)], idxb.at[p, 0], isem[p]),
                    pltpu.make_async_copy(
                        dst_hbm.at[pl.ds(e0, CH)], idxb.at[p, 1], isem[p]))

        def gather_copies(p):
            return (pltpu.make_async_copy(
                        h_hbm.at[idxb.at[p, 0]], hrows.at[p], gsem[p]),
                    pltpu.make_async_copy(
                        ts_hbm.at[idxb.at[p, 0]], tsr.at[p], gsem[p]),
                    pltpu.make_async_copy(
                        td_hbm.at[idxb.at[p, 1]], tdr.at[p], gsem[p]))

        @pl.loop(0, CH)
        def _(e):
            wrows[e] = zero16
            for j in range(8):
                hrows[0, e, pl.ds(j * 16, 16)] = zero16

        r0 = sid * STRIPE
        for k2 in range(STRIPE // CH):
            rr = r0 + k2 * CH
            pltpu.sync_copy(hrows.at[0], accn.at[pl.ds(rr, CH)])
            pltpu.sync_copy(wrows, accd.at[pl.ds(rr, CH)])
        pltpu.sync_copy(hrows.at[0, pl.ds(0, STRIPE - CH * (STRIPE // CH))],
                        accn.at[pl.ds(r0 + CH * (STRIPE // CH),
                                      STRIPE - CH * (STRIPE // CH))])
        pltpu.sync_copy(wrows.at[pl.ds(0, STRIPE - CH * (STRIPE // CH))],
                        accd.at[pl.ds(r0 + CH * (STRIPE // CH),
                                      STRIPE - CH * (STRIPE // CH))])
        plsc.subcore_barrier()

        for c in idx_copies(0, base):
            c.start()
        for c in idx_copies(0, base):
            c.wait()
        for c in gather_copies(0):
            c.start()
        for c in idx_copies(1, base + CH):
            c.start()

        @pl.loop(0, CPW, step=2)
        def _(g):
            for u in range(2):
                p = u
                q = 1 - u
                gg = g + u
                for c in idx_copies(q, 0):
                    c.wait()
                for c in gather_copies(q):
                    c.start()
                for c in gather_copies(p):
                    c.wait()

                @plsc.parallel_loop(0, CH, unroll=8)
                def _(e):
                    t = tsr[p, e] + tdr[p, e]
                    w16 = jnp.exp(jnp.maximum(t, 0.2 * t))
                    wrows[e] = w16
                    for j in range(8):
                        sp = jnp.take(w16, jnp.full((16,), j, jnp.int32))
                        hrows[p, e, pl.ds(j * 16, 16)] = (
                            hrows[p, e, pl.ds(j * 16, 16)] * sp)

                pltpu.sync_copy(hrows.at[p], accn.at[idxb.at[p, 1]], add=True)
                pltpu.sync_copy(wrows, accd.at[idxb.at[p, 1]], add=True)
                for c in idx_copies(p, base + (gg + 2) * CH):
                    c.start()

        for c in gather_copies(0):
            c.wait()
        for c in idx_copies(1, 0):
            c.wait()

        plsc.subcore_barrier()
        pltpu.sync_copy(accn.at[pl.ds(r0, STRIPE)],
                        on_hbm.at[cid, pl.ds(r0, STRIPE)])
        pltpu.sync_copy(accd.at[pl.ds(r0, STRIPE)],
                        od_hbm.at[cid, pl.ds(r0, STRIPE)])

    return k(h, ts, td, srcp, dstp)


def _alpha_mats(a_s, a_d, heads, per):
    if heads > 1:
        eye = jnp.eye(heads, dtype=jnp.float32)
        As8 = (a_s[0][:, :, None] * eye[:, None, :]).reshape(heads * per, heads)
        Ad8 = (a_d[0][:, :, None] * eye[:, None, :]).reshape(heads * per, heads)
        As16 = jnp.concatenate([As8, As8], axis=1)
        Ad16 = jnp.concatenate([Ad8, Ad8], axis=1)
        S = jnp.concatenate(
            [jnp.kron(eye, jnp.ones((1, per), jnp.float32)),
             jnp.zeros((8, heads * per), jnp.float32)], axis=0)
    else:
        As16 = jnp.broadcast_to(a_s[0, 0][:, None], (per, 16))
        Ad16 = jnp.broadcast_to(a_d[0, 0][:, None], (per, 16))
        S = jnp.concatenate(
            [jnp.ones((1, per), jnp.float32),
             jnp.zeros((15, per), jnp.float32)], axis=0)
    return As16, Ad16, S


def kernel(x, edge_index, W1, a_src1, a_dst1, b1, W2, a_src2, a_dst2, b2):
    loops = jnp.arange(N, dtype=jnp.int32)
    pad = ETOT - (edge_index.shape[1] + N)
    srcp = jnp.concatenate(
        [edge_index[0], loops, jnp.zeros((pad + 2 * CH,), jnp.int32)])
    dstp = jnp.concatenate(
        [edge_index[1], loops, jnp.full((pad,), N, jnp.int32),
         jnp.zeros((2 * CH,), jnp.int32)])

    As1, Ad1, S1 = _alpha_mats(a_src1, a_dst1, 8, 16)
    As2, Ad2, S2 = _alpha_mats(a_src2, a_dst2, 1, 128)

    xp = jnp.pad(x, ((0, NPAD - N), (0, 0)))
    h1, ts1, td1 = _dense_front(xp, W1, As1, Ad1)
    on1, od1 = _sc_edge_pass(h1, ts1, td1, srcp, dstp)
    h2in = _dense_finish(on1, od1, S1, b1.reshape(1, 128), True)
    h2, ts2, td2 = _dense_front(h2in, W2, As2, Ad2)
    on2, od2 = _sc_edge_pass(h2, ts2, td2, srcp, dstp)
    out = _dense_finish(on2, od2, S2, b2.reshape(1, 128), False)
    return out[:N]


# speedup vs baseline: 1.3204x; 1.3204x over previous
import functools

import jax
import jax.numpy as jnp
from jax import lax
from jax.experimental import pallas as pl
from jax.experimental.pallas import tpu as pltpu
from jax.experimental.pallas import tpu_sc as plsc

N = 10000
NPAD = 10240
STRIPE = NPAD // 16
NW = 32
CH = 96
CPW = 108
EPW = CPW * CH
ETOT = NW * EPW
ETOT2 = ETOT + 2 * CH
HIGH = lax.Precision.HIGHEST


def _dense_front(xp, W, As16, Ad16):
    BLK = 512

    def body(x_ref, w_ref, as_ref, ad_ref, h_ref, ts_ref, td_ref):
        h = jnp.dot(x_ref[...], w_ref[...], precision=HIGH,
                    preferred_element_type=jnp.float32)
        h_ref[...] = h
        ts_ref[...] = jnp.dot(h, as_ref[...], precision=HIGH,
                              preferred_element_type=jnp.float32)
        td_ref[...] = jnp.dot(h, ad_ref[...], precision=HIGH,
                              preferred_element_type=jnp.float32)

    return pl.pallas_call(
        body,
        grid=(NPAD // BLK,),
        in_specs=[
            pl.BlockSpec((BLK, 128), lambda i: (i, 0)),
            pl.BlockSpec((128, 128), lambda i: (0, 0)),
            pl.BlockSpec((128, 16), lambda i: (0, 0)),
            pl.BlockSpec((128, 16), lambda i: (0, 0)),
        ],
        out_specs=[
            pl.BlockSpec((BLK, 128), lambda i: (i, 0)),
            pl.BlockSpec((BLK, 16), lambda i: (i, 0)),
            pl.BlockSpec((BLK, 16), lambda i: (i, 0)),
        ],
        out_shape=[
            jax.ShapeDtypeStruct((NPAD, 128), jnp.float32),
            jax.ShapeDtypeStruct((NPAD, 16), jnp.float32),
            jax.ShapeDtypeStruct((NPAD, 16), jnp.float32),
        ],
    )(xp, W, As16, Ad16)


def _dense_finish(onum, oden, S, b2d, do_elu):
    BLK = 512

    def body(n_ref, d_ref, s_ref, b_ref, o_ref):
        num = n_ref[0] + n_ref[1]
        den16 = d_ref[0] + d_ref[1]
        den = jnp.dot(den16, s_ref[...], precision=HIGH,
                      preferred_element_type=jnp.float32)
        o = num / (den + 1e-16) + b_ref[...]
        if do_elu:
            o = jnp.where(o > 0.0, o, jnp.exp(o) - 1.0)
        o_ref[...] = o

    return pl.pallas_call(
        body,
        grid=(NPAD // BLK,),
        in_specs=[
            pl.BlockSpec((2, BLK, 128), lambda i: (0, i, 0)),
            pl.BlockSpec((2, BLK, 16), lambda i: (0, i, 0)),
            pl.BlockSpec((16, 128), lambda i: (0, 0)),
            pl.BlockSpec((1, 128), lambda i: (0, 0)),
        ],
        out_specs=pl.BlockSpec((BLK, 128), lambda i: (i, 0)),
        out_shape=jax.ShapeDtypeStruct((NPAD, 128), jnp.float32),
    )(onum, oden, S, b2d)


def _sc_edge_pass(h, ts, td, srcp, dstp):
    mesh = plsc.VectorSubcoreMesh(core_axis_name="c", subcore_axis_name="s")

    @functools.partial(
        pl.kernel,
        compiler_params=pltpu.CompilerParams(
            needs_layout_passes=False, use_tc_tiling_on_sc=False),
        out_type=(
            jax.ShapeDtypeStruct((2, NPAD, 128), jnp.float32),
            jax.ShapeDtypeStruct((2, NPAD, 16), jnp.float32),
        ),
        mesh=mesh,
        scratch_types=[
            pltpu.VMEM((2, 2, CH), jnp.int32),
            pltpu.VMEM((2, CH, 128), jnp.float32),
            pltpu.VMEM((2, CH, 16), jnp.float32),
            pltpu.VMEM((2, CH, 16), jnp.float32),
            pltpu.VMEM((2, CH, 16), jnp.float32),
            pltpu.VMEM((2, 1, CH), jnp.int32),
            pltpu.VMEM_SHARED((NPAD, 128), jnp.float32),
            pltpu.VMEM_SHARED((NPAD, 16), jnp.float32),
            pltpu.SemaphoreType.DMA,
            pltpu.SemaphoreType.DMA,
            pltpu.SemaphoreType.DMA,
            pltpu.SemaphoreType.DMA,
            pltpu.SemaphoreType.DMA,
            pltpu.SemaphoreType.DMA,
        ],
    )
    def k(h_hbm, ts_hbm, td_hbm, src_hbm, dst_hbm, on_hbm, od_hbm,
          idxb, hrows, tsr, tdr, wrows, dstx, accn, accd,
          isem0, isem1, gsem0, gsem1, ssem0, ssem1):
        cid = lax.axis_index("c")
        sid = lax.axis_index("s")
        wid = cid * 16 + sid
        base = wid * EPW
        zero16 = jnp.zeros((16,), jnp.float32)
        isem = (isem0, isem1)
        gsem = (gsem0, gsem1)
        ssem = (ssem0, ssem1)

        def idx_copies(p, e0):
            return (pltpu.make_async_copy(
                        src_hbm.at[pl.ds(e0, CH)], idxb.at[p, 0], isem[p]),
                    pltpu.make_async_copy(
                        dst_hbm.at[pl.ds(e0, CH)], idxb.at[p, 1], isem[p]))

        def gather_copies(p):
            return (pltpu.make_async_copy(
                        h_hbm.at[idxb.at[p, 0]], hrows.at[p], gsem[p]),
                    pltpu.make_async_copy(
                        ts_hbm.at[idxb.at[p, 0]], tsr.at[p], gsem[p]),
                    pltpu.make_async_copy(
                        td_hbm.at[idxb.at[p, 1]], tdr.at[p], gsem[p]))

        def scatter_copies(p):
            return (pltpu.make_async_copy(
                        hrows.at[p], accn.at[dstx.at[p, 0]], ssem[p]),
                    pltpu.make_async_copy(
                        wrows.at[p], accd.at[dstx.at[p, 0]], ssem[p]))

        @pl.loop(0, CH)
        def _(e):
            wrows[0, e] = zero16
            for j in range(8):
                hrows[0, e, pl.ds(j * 16, 16)] = zero16

        r0 = sid * STRIPE
        for k2 in range(STRIPE // CH):
            rr = r0 + k2 * CH
            pltpu.sync_copy(hrows.at[0], accn.at[pl.ds(rr, CH)])
            pltpu.sync_copy(wrows.at[0], accd.at[pl.ds(rr, CH)])
        pltpu.sync_copy(hrows.at[0, pl.ds(0, STRIPE - CH * (STRIPE // CH))],
                        accn.at[pl.ds(r0 + CH * (STRIPE // CH),
                                      STRIPE - CH * (STRIPE // CH))])
        pltpu.sync_copy(wrows.at[0, pl.ds(0, STRIPE - CH * (STRIPE // CH))],
                        accd.at[pl.ds(r0 + CH * (STRIPE // CH),
                                      STRIPE - CH * (STRIPE // CH))])
        plsc.subcore_barrier()

        for c in idx_copies(0, base):
            c.start()
        for c in idx_copies(0, base):
            c.wait()
        for c in gather_copies(0):
            c.start()
        for c in idx_copies(1, base + CH):
            c.start()

        @pl.loop(0, CPW, step=2)
        def _(g):
            for u in range(2):
                p = u
                q = 1 - u
                gg = g + u
                for c in idx_copies(q, 0):
                    c.wait()
                if u == 1:
                    for c in scatter_copies(q):
                        c.wait()
                else:
                    @pl.when(g >= 1)
                    def _():
                        for c in scatter_copies(q):
                            c.wait()
                for c in gather_copies(q):
                    c.start()
                for c in gather_copies(p):
                    c.wait()

                for k6 in range(CH // 16):
                    dstx[p, 0, pl.ds(k6 * 16, 16)] = (
                        idxb[p, 1, pl.ds(k6 * 16, 16)])

                @plsc.parallel_loop(0, CH, unroll=4)
                def _(e):
                    t = tsr[p, e] + tdr[p, e]
                    w16 = jnp.exp(jnp.maximum(t, 0.2 * t))
                    wrows[p, e] = w16
                    for j in range(8):
                        sp = jnp.take(w16, jnp.full((16,), j, jnp.int32))
                        hrows[p, e, pl.ds(j * 16, 16)] = (
                            hrows[p, e, pl.ds(j * 16, 16)] * sp)

                for c in scatter_copies(p):
                    c.start(add=True)
                for c in idx_copies(p, base + (gg + 2) * CH):
                    c.start()

        for c in gather_copies(0):
            c.wait()
        for c in idx_copies(1, 0):
            c.wait()
        for c in scatter_copies(1):
            c.wait()

        plsc.subcore_barrier()
        pltpu.sync_copy(accn.at[pl.ds(r0, STRIPE)],
                        on_hbm.at[cid, pl.ds(r0, STRIPE)])
        pltpu.sync_copy(accd.at[pl.ds(r0, STRIPE)],
                        od_hbm.at[cid, pl.ds(r0, STRIPE)])

    return k(h, ts, td, srcp, dstp)


def _alpha_mats(a_s, a_d, heads, per):
    if heads > 1:
        eye = jnp.eye(heads, dtype=jnp.float32)
        As8 = (a_s[0][:, :, None] * eye[:, None, :]).reshape(heads * per, heads)
        Ad8 = (a_d[0][:, :, None] * eye[:, None, :]).reshape(heads * per, heads)
        As16 = jnp.concatenate([As8, As8], axis=1)
        Ad16 = jnp.concatenate([Ad8, Ad8], axis=1)
        S = jnp.concatenate(
            [jnp.kron(eye, jnp.ones((1, per), jnp.float32)),
             jnp.zeros((8, heads * per), jnp.float32)], axis=0)
    else:
        As16 = jnp.broadcast_to(a_s[0, 0][:, None], (per, 16))
        Ad16 = jnp.broadcast_to(a_d[0, 0][:, None], (per, 16))
        S = jnp.concatenate(
            [jnp.ones((1, per), jnp.float32),
             jnp.zeros((15, per), jnp.float32)], axis=0)
    return As16, Ad16, S


def kernel(x, edge_index, W1, a_src1, a_dst1, b1, W2, a_src2, a_dst2, b2):
    loops = jnp.arange(N, dtype=jnp.int32)
    pad = ETOT - (edge_index.shape[1] + N)
    srcp = jnp.concatenate(
        [edge_index[0], loops, jnp.zeros((pad + 2 * CH,), jnp.int32)])
    dstp = jnp.concatenate(
        [edge_index[1], loops, jnp.full((pad,), N, jnp.int32),
         jnp.zeros((2 * CH,), jnp.int32)])

    As1, Ad1, S1 = _alpha_mats(a_src1, a_dst1, 8, 16)
    As2, Ad2, S2 = _alpha_mats(a_src2, a_dst2, 1, 128)

    xp = jnp.pad(x, ((0, NPAD - N), (0, 0)))
    h1, ts1, td1 = _dense_front(xp, W1, As1, Ad1)
    on1, od1 = _sc_edge_pass(h1, ts1, td1, srcp, dstp)
    h2in = _dense_finish(on1, od1, S1, b1.reshape(1, 128), True)
    h2, ts2, td2 = _dense_front(h2in, W2, As2, Ad2)
    on2, od2 = _sc_edge_pass(h2, ts2, td2, srcp, dstp)
    out = _dense_finish(on2, od2, S2, b2.reshape(1, 128), False)
    return out[:N]
